# baseline (speedup 1.0000x reference)
import functools

import jax
import jax.numpy as jnp
import numpy as np
from jax import lax
from jax.experimental import pallas as pl
from jax.experimental.pallas import tpu as pltpu
from jax.experimental.pallas import tpu_sc as plsc

B, F, D, V, DIN = 4096, 26, 64, 1000000, 13
NF = F + 1
NFP = 32
NPAIR = NF * (NF - 1) // 2

_S = np.zeros((NPAIR, NFP * NF), np.float32)
_p = 0
for _n in range(NF):
  for _m in range(_n + 1, NF):
    _S[_p, _n * NFP + _m] = 1.0
    _p += 1

NC, NS = 2, 16
NW = NC * NS
TOTAL = B * F
PER_WB = B // NW
CHUNK_B = 32
NCHUNK = PER_WB // CHUNK_B


PER_W = PER_WB * F
CHUNK = CHUNK_B * F


def _sc_gather_body(table_hbm, idx_hbm, out_hbm, idx2d_v, rows_v, sem):
  wid = lax.axis_index("s") * NC + lax.axis_index("c")
  base = wid * PER_WB
  pltpu.sync_copy(idx_hbm.at[pl.ds(base, PER_WB)], idx2d_v)
  for ci in range(NCHUNK):
    copies = []
    for i in range(CHUNK_B):
      row = ci * CHUNK_B + i
      copies.append(pltpu.async_copy(
          table_hbm.at[idx2d_v.at[row]],
          rows_v.at[pl.ds(i * F, F)], sem))
    for c in copies:
      c.wait()
    pltpu.sync_copy(rows_v, out_hbm.at[pl.ds(base * F + ci * CHUNK, CHUNK)])


def _sc_gather(table, idx2d):
  mesh = plsc.VectorSubcoreMesh(core_axis_name="c", subcore_axis_name="s")
  fn = functools.partial(
      pl.kernel,
      mesh=mesh,
      out_type=jax.ShapeDtypeStruct((TOTAL, D), jnp.float32),
      scratch_types=[
          pltpu.VMEM((PER_WB, F), jnp.int32),
          pltpu.VMEM((CHUNK, D), jnp.float32),
          pltpu.SemaphoreType.DMA,
      ],
      compiler_params=pltpu.CompilerParams(use_tc_tiling_on_sc=False),
  )(_sc_gather_body)
  return fn(table, idx2d)


BT = 128
GRID = B // BT


def _tc_body(xdT_ref, emb_ref, dw1T_ref, db1_ref, dw2T_ref, db2_ref,
             dw3T_ref, db3_ref, ow1aT_ref, w2T_ref, ob1_ref, ow2T_ref,
             ob2_ref, ow3T_ref, ob3_ref, out_ref):
  f32 = jnp.float32
  dot = lambda a, b: jax.lax.dot_general(
      a, b, (((1,), (0,)), ((), ())), preferred_element_type=f32)
  h = jnp.maximum(dot(dw1T_ref[...], xdT_ref[...]) + db1_ref[...], 0.0)
  h = jnp.maximum(dot(dw2T_ref[...], h) + db2_ref[...], 0.0)
  doT = jnp.maximum(dot(dw3T_ref[...], h) + db3_ref[...], 0.0)
  embT = emb_ref[...].T
  cT = jnp.concatenate(
      [doT, embT, jnp.zeros(((NFP - NF) * D, BT), f32)], axis=0)
  c3 = cT.reshape(NFP, D, BT)
  gs = []
  for n in range(NF):
    prod = c3 * c3[n][None]
    gs.append(jnp.sum(prod, axis=1))
  g = jnp.concatenate(gs, axis=0)
  acc = dot(w2T_ref[...], g) + dot(ow1aT_ref[...], doT) + ob1_ref[...]
  h = jnp.maximum(acc, 0.0)
  h = jnp.maximum(dot(ow2T_ref[...], h) + ob2_ref[...], 0.0)
  out_ref[...] = dot(ow3T_ref[...], h) + ob3_ref[...]


def _tc_main(xdT, emb2d, dw1T, db1, dw2T, db2, dw3T, db3, ow1aT, w2T, ob1,
             ow2T, ob2, ow3T, ob3, *, interpret=False):
  full = lambda shape: pl.BlockSpec(shape, lambda i: (0,) * len(shape))
  return pl.pallas_call(
      _tc_body,
      grid=(GRID,),
      in_specs=[
          pl.BlockSpec((16, BT), lambda i: (0, i)),
          pl.BlockSpec((BT, F * D), lambda i: (i, 0)),
          full((512, 16)), full((512, 1)),
          full((256, 512)), full((256, 1)),
          full((D, 256)), full((D, 1)),
          full((512, D)), full((512, NF * NFP)),
          full((512, 1)), full((256, 512)), full((256, 1)),
          full((8, 256)), full((8, 1)),
      ],
      out_specs=pl.BlockSpec((8, BT), lambda i: (0, i)),
      out_shape=jax.ShapeDtypeStruct((8, B), jnp.float32),
      interpret=interpret,
  )(xdT, emb2d, dw1T, db1, dw2T, db2, dw3T, db3, ow1aT, w2T, ob1, ow2T,
    ob2, ow3T, ob3)


def kernel(dense_features, sparse_indices, table, dw1, db1, dw2, db2, dw3,
           db3, ow1, ob1, ow2, ob2, ow3, ob3):
  idx2d = sparse_indices.astype(jnp.int32)
  xdT = jnp.pad(dense_features, ((0, 0), (0, 16 - DIN))).T
  dw1T = jnp.pad(dw1, ((0, 16 - DIN), (0, 0))).T
  ow1aT = ow1[:D].T
  w2T = jnp.dot(ow1[D:].T, jnp.asarray(_S))
  ow3T = jnp.pad(ow3, ((0, 0), (0, 7))).T
  ob3T = jnp.pad(ob3, ((0, 7),)).reshape(8, 1)
  col = lambda b: b.reshape(-1, 1)

  emb = _sc_gather(table, idx2d)
  emb2d = emb.reshape(B, F * D)

  out = _tc_main(xdT, emb2d, dw1T.astype(jnp.float32), col(db1), dw2.T,
                 col(db2), dw3.T, col(db3), ow1aT, w2T, col(ob1), ow2.T,
                 col(ob2), ow3T, ob3T)
  return out[0].reshape(B, 1)
